# baseline (speedup 1.0000x reference)
import functools

import jax
import jax.numpy as jnp
from jax import lax
from jax.experimental import pallas as pl
from jax.experimental.pallas import tpu as pltpu
from jax.experimental.pallas import tpu_sc as plsc

BATCH = 16384
ID_DIM = 32
N_CONT = 16
N_ROWS = 1000000
TAIL0 = (N_ROWS // 128) * 128

_NC = 2
_NS = 16
_NW = _NC * _NS
_BPW = BATCH // _NW
_FLUSH = 128
_NFLUSH = _BPW // _FLUSH
_NBUF = 8


def _do_table(tab, ids_v, tail_v, out_hbm, out_v, bufs, sems, sem_out, base):
    rows_lo = lax.iota(jnp.int32, 16)
    rows_hi = rows_lo + 16
    ngroups = _FLUSH // 16

    def idks_of(ids16):
        return [jnp.max(jnp.where(rows_lo == k, ids16, 0)) for k in range(16)]

    def fire(idk, slot):
        tc = jnp.where(idk >= TAIL0, 0, lax.shift_right_logical(idk, 7))
        return pltpu.async_copy(
            tab.at[:, pl.ds(tc * 128, 128)], bufs[slot], sems[slot])

    def extract(idk, slot, j):
        buf = bufs[slot]
        lane = jnp.full((16,), idk & 127, jnp.int32)
        tlane = jnp.full((16,), jnp.clip(idk - TAIL0, 0, 127), jnp.int32)
        is_tail = jnp.full((16,), idk >= TAIL0, jnp.bool_)
        v_lo = jnp.where(is_tail,
                         plsc.load_gather(tail_v, [rows_lo, tlane]),
                         plsc.load_gather(buf, [rows_lo, lane]))
        v_hi = jnp.where(is_tail,
                         plsc.load_gather(tail_v, [rows_hi, tlane]),
                         plsc.load_gather(buf, [rows_hi, lane]))
        jsplat = jnp.full((16,), j, jnp.int32)
        plsc.store_scatter(out_v, [jsplat, rows_lo], v_lo)
        plsc.store_scatter(out_v, [jsplat, rows_hi], v_hi)

    def flush_body(f, carry):
        idks_cur = idks_of(ids_v[pl.ds(f * _FLUSH, 16)])
        copies = [None] * _FLUSH
        for k in range(_NBUF):
            copies[k] = fire(idks_cur[k], k)
        for g in range(ngroups):
            if g + 1 < ngroups:
                idks_next = idks_of(ids_v[pl.ds(f * _FLUSH + (g + 1) * 16, 16)])
            else:
                idks_next = None
            for k in range(16):
                gk = g * 16 + k
                copies[gk].wait()
                extract(idks_cur[k], gk % _NBUF, gk)
                nk = gk + _NBUF
                if nk < _FLUSH:
                    if nk < (g + 1) * 16:
                        idk_n = idks_cur[nk - g * 16]
                    else:
                        idk_n = idks_next[nk - (g + 1) * 16]
                    copies[nk] = fire(idk_n, nk % _NBUF)
            idks_cur = idks_next
        pltpu.async_copy(
            out_v, out_hbm.at[pl.ds(base + f * _FLUSH, _FLUSH)], sem_out).wait()
        return carry

    lax.fori_loop(0, _NFLUSH, flush_body, 0)


def _sc_gather_body(ut, uids, tailu, it, iids, taili, u_out, i_out,
                    idsu_v, idsi_v, tailu_v, taili_v,
                    bufs, out_v, sems, sem_out, sem_in):
    wid = lax.axis_index("s") * _NC + lax.axis_index("c")
    base = wid * _BPW
    pltpu.sync_copy(uids.at[pl.ds(base, _BPW)], idsu_v)
    pltpu.sync_copy(iids.at[pl.ds(base, _BPW)], idsi_v)
    pltpu.sync_copy(tailu, tailu_v)
    pltpu.sync_copy(taili, taili_v)
    _do_table(ut, idsu_v, tailu_v, u_out, out_v, bufs, sems, sem_out, base)
    _do_table(it, idsi_v, taili_v, i_out, out_v, bufs, sems, sem_out, base)


def _sc_gather(U_table, user_ids, I_table, item_ids):
    ut = U_table.T
    it = I_table.T
    tailu = jnp.pad(U_table[TAIL0:].T, ((0, 0), (0, 128 - (N_ROWS - TAIL0))))
    taili = jnp.pad(I_table[TAIL0:].T, ((0, 0), (0, 128 - (N_ROWS - TAIL0))))
    mesh = plsc.VectorSubcoreMesh(core_axis_name="c", subcore_axis_name="s",
                                  num_cores=_NC, num_subcores=_NS)
    f = pl.kernel(
        _sc_gather_body,
        out_type=[jax.ShapeDtypeStruct((BATCH, ID_DIM), jnp.float32),
                  jax.ShapeDtypeStruct((BATCH, ID_DIM), jnp.float32)],
        mesh=mesh,
        scratch_types=[
            pltpu.VMEM((_BPW,), jnp.int32),
            pltpu.VMEM((_BPW,), jnp.int32),
            pltpu.VMEM((ID_DIM, 128), jnp.float32),
            pltpu.VMEM((ID_DIM, 128), jnp.float32),
            [pltpu.VMEM((ID_DIM, 128), jnp.float32) for _ in range(_NBUF)],
            pltpu.VMEM((_FLUSH, ID_DIM), jnp.float32),
            [pltpu.SemaphoreType.DMA for _ in range(_NBUF)],
            pltpu.SemaphoreType.DMA,
            pltpu.SemaphoreType.DMA,
        ],
        compiler_params=pltpu.CompilerParams(use_tc_tiling_on_sc=True,
                                             needs_layout_passes=False),
    )
    return f(ut, user_ids, tailu, it, item_ids, taili)


def _tower(e, c, w1a, w1b, b1, w2, b2, w3, b3):
    hp = jax.lax.Precision.DEFAULT
    h = (jnp.dot(e, w1a, preferred_element_type=jnp.float32, precision=hp)
         + jnp.dot(c, w1b, preferred_element_type=jnp.float32, precision=hp)
         + b1)
    h = jnp.maximum(h, 0.0)
    h = jnp.maximum(
        jnp.dot(h, w2, preferred_element_type=jnp.float32, precision=hp) + b2, 0.0)
    return jnp.dot(h, w3, preferred_element_type=jnp.float32, precision=hp) + b3


def _tc_mlp_body(ue, uc, ie, ic,
                 uw1a, uw1b, ub1, uw2, ub2, uw3, ub3,
                 iw1a, iw1b, ib1, iw2, ib2, iw3, ib3, out):
    u = _tower(ue[...], uc[...], uw1a[...], uw1b[...], ub1[...],
               uw2[...], ub2[...], uw3[...], ub3[...])
    v = _tower(ie[...], ic[...], iw1a[...], iw1b[...], ib1[...],
               iw2[...], ib2[...], iw3[...], ib3[...])
    out[...] = jnp.sum(u * v, axis=1)


def _tc_mlp(ue, uc, ie, ic, weights):
    grid = 32
    rows = BATCH // grid
    bspec_rows = lambda d: pl.BlockSpec((rows, d), lambda i: (i, 0))
    full = lambda a: pl.BlockSpec(a.shape, lambda i: (0,) * a.ndim)
    in_specs = [bspec_rows(ID_DIM), bspec_rows(N_CONT),
                bspec_rows(ID_DIM), bspec_rows(N_CONT)]
    in_specs += [full(w) for w in weights]
    return pl.pallas_call(
        _tc_mlp_body,
        grid=(grid,),
        in_specs=in_specs,
        out_specs=pl.BlockSpec((rows,), lambda i: (i,)),
        out_shape=jax.ShapeDtypeStruct((BATCH,), jnp.float32),
    )(ue, uc, ie, ic, *weights)


def kernel(user_ids, user_cont, item_ids, item_cont, U_table, I_table,
           Uw1, Ub1, Uw2, Ub2, Uw3, Ub3,
           Iw1, Ib1, Iw2, Ib2, Iw3, Ib3):
    ue, ie = _sc_gather(U_table, user_ids, I_table, item_ids)
    weights = (
        Uw1[:ID_DIM], Uw1[ID_DIM:], Ub1.reshape(1, -1),
        Uw2, Ub2.reshape(1, -1), Uw3, Ub3.reshape(1, -1),
        Iw1[:ID_DIM], Iw1[ID_DIM:], Ib1.reshape(1, -1),
        Iw2, Ib2.reshape(1, -1), Iw3, Ib3.reshape(1, -1),
    )
    return _tc_mlp(ue, user_cont, ie, item_cont, weights)


# speedup vs baseline: 1.0238x; 1.0238x over previous
import functools

import jax
import jax.numpy as jnp
from jax import lax
from jax.experimental import pallas as pl
from jax.experimental.pallas import tpu as pltpu
from jax.experimental.pallas import tpu_sc as plsc

BATCH = 16384
ID_DIM = 32
N_CONT = 16
N_ROWS = 1000000
TAIL0 = (N_ROWS // 128) * 128

_NC = 2
_NS = 16
_NW = _NC * _NS
_BPW = BATCH // _NW
_FLUSH = 128
_NFLUSH = _BPW // _FLUSH
_NBUF = 8


def _do_table(tab, ids_v, tail_v, out_hbm, out_v, bufs, sems, sem_out, base):
    rows_lo = lax.iota(jnp.int32, 16)
    rows_hi = rows_lo + 16
    ngroups = _FLUSH // 16

    def idks_of(ids16):
        return [jnp.max(jnp.where(rows_lo == k, ids16, 0)) for k in range(16)]

    def fire(idk, slot):
        tc = jnp.where(idk >= TAIL0, 0, lax.shift_right_logical(idk, 7))
        return pltpu.async_copy(
            tab.at[:, pl.ds(tc * 128, 128)], bufs[slot], sems[slot])

    def extract(idk, slot, j):
        buf = bufs[slot]
        lane = jnp.full((16,), idk & 127, jnp.int32)
        tlane = jnp.full((16,), jnp.clip(idk - TAIL0, 0, 127), jnp.int32)
        is_tail = jnp.full((16,), idk >= TAIL0, jnp.bool_)
        v_lo = jnp.where(is_tail,
                         plsc.load_gather(tail_v, [rows_lo, tlane]),
                         plsc.load_gather(buf, [rows_lo, lane]))
        v_hi = jnp.where(is_tail,
                         plsc.load_gather(tail_v, [rows_hi, tlane]),
                         plsc.load_gather(buf, [rows_hi, lane]))
        jsplat = jnp.full((16,), j, jnp.int32)
        plsc.store_scatter(out_v, [jsplat, rows_lo], v_lo)
        plsc.store_scatter(out_v, [jsplat, rows_hi], v_hi)

    def flush_body(f, carry):
        idks_cur = idks_of(ids_v[pl.ds(f * _FLUSH, 16)])
        copies = [None] * _FLUSH
        for k in range(_NBUF):
            copies[k] = fire(idks_cur[k], k)
        for g in range(ngroups):
            if g + 1 < ngroups:
                idks_next = idks_of(ids_v[pl.ds(f * _FLUSH + (g + 1) * 16, 16)])
            else:
                idks_next = None
            for k in range(16):
                gk = g * 16 + k
                copies[gk].wait()
                extract(idks_cur[k], gk % _NBUF, gk)
                nk = gk + _NBUF
                if nk < _FLUSH:
                    if nk < (g + 1) * 16:
                        idk_n = idks_cur[nk - g * 16]
                    else:
                        idk_n = idks_next[nk - (g + 1) * 16]
                    copies[nk] = fire(idk_n, nk % _NBUF)
            idks_cur = idks_next
        pltpu.async_copy(
            out_v, out_hbm.at[pl.ds(base + f * _FLUSH, _FLUSH)], sem_out).wait()
        return carry

    lax.fori_loop(0, _NFLUSH, flush_body, 0)


def _sc_gather_body(ut, uids, tailu, it, iids, taili, u_out, i_out,
                    idsu_v, idsi_v, tailu_v, taili_v,
                    bufs, out_v, sems, sem_out, sem_in):
    wid = lax.axis_index("s") * _NC + lax.axis_index("c")
    base = wid * _BPW
    pltpu.sync_copy(uids.at[pl.ds(base, _BPW)], idsu_v)
    pltpu.sync_copy(iids.at[pl.ds(base, _BPW)], idsi_v)
    pltpu.sync_copy(tailu, tailu_v)
    pltpu.sync_copy(taili, taili_v)
    _do_table(ut, idsu_v, tailu_v, u_out, out_v, bufs, sems, sem_out, base)
    _do_table(it, idsi_v, taili_v, i_out, out_v, bufs, sems, sem_out, base)


def _sc_gather(U_table, user_ids, I_table, item_ids):
    ut = U_table.T
    it = I_table.T
    tailu = jnp.pad(U_table[TAIL0:].T, ((0, 0), (0, 128 - (N_ROWS - TAIL0))))
    taili = jnp.pad(I_table[TAIL0:].T, ((0, 0), (0, 128 - (N_ROWS - TAIL0))))
    mesh = plsc.VectorSubcoreMesh(core_axis_name="c", subcore_axis_name="s",
                                  num_cores=_NC, num_subcores=_NS)
    f = pl.kernel(
        _sc_gather_body,
        out_type=[jax.ShapeDtypeStruct((BATCH, ID_DIM), jnp.float32),
                  jax.ShapeDtypeStruct((BATCH, ID_DIM), jnp.float32)],
        mesh=mesh,
        scratch_types=[
            pltpu.VMEM((_BPW,), jnp.int32),
            pltpu.VMEM((_BPW,), jnp.int32),
            pltpu.VMEM((ID_DIM, 128), jnp.float32),
            pltpu.VMEM((ID_DIM, 128), jnp.float32),
            [pltpu.VMEM((ID_DIM, 128), jnp.float32) for _ in range(_NBUF)],
            pltpu.VMEM((_FLUSH, ID_DIM), jnp.float32),
            [pltpu.SemaphoreType.DMA for _ in range(_NBUF)],
            pltpu.SemaphoreType.DMA,
            pltpu.SemaphoreType.DMA,
        ],
        compiler_params=pltpu.CompilerParams(use_tc_tiling_on_sc=True,
                                             needs_layout_passes=False),
    )
    return f(ut, user_ids, tailu, it, item_ids, taili)


def _tower(e, c, w1a, w1b, b1, w2, b2, w3, b3):
    hp = jax.lax.Precision.DEFAULT
    h = (jnp.dot(e, w1a, preferred_element_type=jnp.float32, precision=hp)
         + jnp.dot(c, w1b, preferred_element_type=jnp.float32, precision=hp)
         + b1)
    h = jnp.maximum(h, 0.0)
    h = jnp.maximum(
        jnp.dot(h, w2, preferred_element_type=jnp.float32, precision=hp) + b2, 0.0)
    return jnp.dot(h, w3, preferred_element_type=jnp.float32, precision=hp) + b3


def _tc_mlp_body(ue, uc, ie, ic,
                 uw1a, uw1b, ub1, uw2, ub2, uw3, ub3,
                 iw1a, iw1b, ib1, iw2, ib2, iw3, ib3, out):
    u = _tower(ue[...], uc[...], uw1a[...], uw1b[...], ub1[...],
               uw2[...], ub2[...], uw3[...], ub3[...])
    v = _tower(ie[...], ic[...], iw1a[...], iw1b[...], ib1[...],
               iw2[...], ib2[...], iw3[...], ib3[...])
    out[...] = jnp.sum(u * v, axis=1)


def _tc_mlp(ue, uc, ie, ic, weights):
    grid = 16
    rows = BATCH // grid
    bspec_rows = lambda d: pl.BlockSpec((rows, d), lambda i: (i, 0))
    full = lambda a: pl.BlockSpec(a.shape, lambda i: (0,) * a.ndim)
    in_specs = [bspec_rows(ID_DIM), bspec_rows(N_CONT),
                bspec_rows(ID_DIM), bspec_rows(N_CONT)]
    in_specs += [full(w) for w in weights]
    return pl.pallas_call(
        _tc_mlp_body,
        grid=(grid,),
        in_specs=in_specs,
        out_specs=pl.BlockSpec((rows,), lambda i: (i,)),
        out_shape=jax.ShapeDtypeStruct((BATCH,), jnp.float32),
    )(ue, uc, ie, ic, *weights)


def kernel(user_ids, user_cont, item_ids, item_cont, U_table, I_table,
           Uw1, Ub1, Uw2, Ub2, Uw3, Ub3,
           Iw1, Ib1, Iw2, Ib2, Iw3, Ib3):
    ue, ie = _sc_gather(U_table, user_ids, I_table, item_ids)
    weights = (
        Uw1[:ID_DIM], Uw1[ID_DIM:], Ub1.reshape(1, -1),
        Uw2, Ub2.reshape(1, -1), Uw3, Ub3.reshape(1, -1),
        Iw1[:ID_DIM], Iw1[ID_DIM:], Ib1.reshape(1, -1),
        Iw2, Ib2.reshape(1, -1), Iw3, Ib3.reshape(1, -1),
    )
    return _tc_mlp(ue, user_cont, ie, item_cont, weights)
